# Initial kernel scaffold; baseline (speedup 1.0000x reference)
import functools

import jax
import jax.numpy as jnp
from jax import lax
from jax.experimental import pallas as pl
from jax.experimental.pallas import tpu as pltpu
from jax.experimental.pallas import tpu_sc as plsc

NPAD = 10240
EPAD = 163840
CW = 64
ER = EPAD // CW
NC, NS = 2, 16
RPT = ER // (NC * NS)
RPT_DEG = ER // NS
RN = NPAD // NS

_MESH = plsc.VectorSubcoreMesh(
    core_axis_name="c", subcore_axis_name="s", num_cores=NC, num_subcores=NS)


def _sc_degrees(src2, dst2):
  @functools.partial(
      pl.kernel,
      out_type=jax.ShapeDtypeStruct((NC, NPAD), jnp.float32),
      mesh=_MESH,
      scratch_types=[
          pltpu.VMEM_SHARED((NPAD,), jnp.float32),
          pltpu.VMEM((RPT_DEG, CW), jnp.int32),
          pltpu.VMEM((CW,), jnp.float32),
          pltpu.VMEM((RN,), jnp.float32),
      ],
  )
  def deg_kernel(src_hbm, dst_hbm, out_hbm, acc, idx, ones, zbuf):
    c = lax.axis_index("c")
    s = lax.axis_index("s")

    def fill(i, _):
      ones[pl.ds(i * 16, 16)] = jnp.full((16,), 1.0, jnp.float32)
      return 0
    lax.fori_loop(0, CW // 16, fill, 0)

    def zfill(i, _):
      zbuf[pl.ds(i * 16, 16)] = jnp.zeros((16,), jnp.float32)
      return 0
    lax.fori_loop(0, RN // 16, zfill, 0)
    pltpu.sync_copy(zbuf, acc.at[pl.ds(s * RN, RN)])
    plsc.subcore_barrier()

    def scan(idx_hbm):
      pltpu.sync_copy(idx_hbm.at[pl.ds(s * RPT_DEG, RPT_DEG)], idx)

      def row(j, _):
        pltpu.sync_copy(ones, acc.at[idx.at[j]], add=True)
        return 0
      lax.fori_loop(0, RPT_DEG, row, 0)

    @pl.when(c == 0)
    def _():
      scan(src_hbm)

    @pl.when(c == 1)
    def _():
      scan(dst_hbm)

    plsc.subcore_barrier()
    pltpu.sync_copy(acc.at[pl.ds(s * RN, RN)], out_hbm.at[c, pl.ds(s * RN, RN)])

  return deg_kernel(src2, dst2)


def _tc_node(featp, w1, w2a, b2, outdeg):
  blk = 512
  grid = NPAD // blk

  def body(x_ref, w1_ref, w2a_ref, b2_ref, od_ref, a1_ref, lp_ref, pk_ref):
    x = x_ref[...]
    s = lax.rsqrt(jnp.clip(od_ref[...], 1.0, None))
    a1 = jnp.dot(x, w1_ref[...], preferred_element_type=jnp.float32) * s
    z = jnp.dot(x, w2a_ref[...], preferred_element_type=jnp.float32) * s
    p = jnp.tanh(z + b2_ref[...])
    a1_ref[...] = a1
    lp_ref[...] = jnp.log(jnp.abs(p))
    negp = jnp.where(p < 0, 1.0, 0.0).astype(jnp.float32)
    pk_ref[...] = negp[:, :64] + 4096.0 * negp[:, 64:]

  return pl.pallas_call(
      body,
      grid=(grid,),
      in_specs=[
          pl.BlockSpec((blk, 128), lambda i: (i, 0)),
          pl.BlockSpec((128, 128), lambda i: (0, 0)),
          pl.BlockSpec((128, 128), lambda i: (0, 0)),
          pl.BlockSpec((1, 128), lambda i: (0, 0)),
          pl.BlockSpec((blk, 1), lambda i: (i, 0)),
      ],
      out_specs=[
          pl.BlockSpec((blk, 128), lambda i: (i, 0)),
          pl.BlockSpec((blk, 128), lambda i: (i, 0)),
          pl.BlockSpec((blk, 64), lambda i: (i, 0)),
      ],
      out_shape=[
          jax.ShapeDtypeStruct((NPAD, 128), jnp.float32),
          jax.ShapeDtypeStruct((NPAD, 128), jnp.float32),
          jax.ShapeDtypeStruct((NPAD, 64), jnp.float32),
      ],
  )(featp, w1, w2a, b2, outdeg)


def _tc_combo(bond_tables):
  def body(bt_ref, ew_ref, lew_ref, pk_ref):
    bt = bt_ref[...]
    t01 = (bt[0][:, None, :] + bt[1][None, :, :]).reshape(64, 128)
    t012 = (t01[:, None, :] + bt[2][None, :, :]).reshape(512, 128)
    ew = (t012[:, None, :] + bt[3][None, :, :]).reshape(4096, 128)
    ew_ref[...] = ew
    lew_ref[...] = jnp.log(jnp.abs(ew))
    neg = jnp.where(ew < 0, 1.0, 0.0).astype(jnp.float32)
    pk_ref[...] = neg[:, :64] + 4096.0 * neg[:, 64:]

  return pl.pallas_call(
      body,
      out_shape=[
          jax.ShapeDtypeStruct((4096, 128), jnp.float32),
          jax.ShapeDtypeStruct((4096, 128), jnp.float32),
          jax.ShapeDtypeStruct((4096, 64), jnp.float32),
      ],
  )(bond_tables)


def _tc_codes(ewT):
  def body(e_ref, c_ref):
    ev = e_ref[...]
    c_ref[...] = ev[0:1] * 512 + ev[1:2] * 64 + ev[2:3] * 8 + ev[3:4]

  return pl.pallas_call(
      body,
      out_shape=jax.ShapeDtypeStruct((1, EPAD), jnp.int32),
  )(ewT)


def _tc_post(hs, ml, pk, v, indeg):
  blk = 512
  grid = NPAD // blk

  def body(hs_ref, ml_ref, pk_ref, v_ref, id_ref, out_ref):
    pkv = pk_ref[...]
    c1 = jnp.floor(pkv * (1.0 / 4096.0))
    c0 = pkv - 4096.0 * c1
    s0 = 1.0 - 2.0 * (c0 - 2.0 * jnp.floor(c0 * 0.5))
    s1 = 1.0 - 2.0 * (c1 - 2.0 * jnp.floor(c1 * 0.5))
    sgn = jnp.concatenate([s0, s1], axis=1)
    hp = jnp.exp(ml_ref[...]) * sgn
    r = hs_ref[...] + jnp.dot(hp, v_ref[...], preferred_element_type=jnp.float32)
    out_ref[...] = r * lax.rsqrt(jnp.clip(id_ref[...], 1.0, None))

  return pl.pallas_call(
      body,
      grid=(grid,),
      in_specs=[
          pl.BlockSpec((blk, 128), lambda i: (i, 0)),
          pl.BlockSpec((blk, 128), lambda i: (i, 0)),
          pl.BlockSpec((blk, 64), lambda i: (i, 0)),
          pl.BlockSpec((128, 128), lambda i: (0, 0)),
          pl.BlockSpec((blk, 1), lambda i: (i, 0)),
      ],
      out_specs=pl.BlockSpec((blk, 128), lambda i: (i, 0)),
      out_shape=jax.ShapeDtypeStruct((NPAD, 128), jnp.float32),
  )(hs, ml, pk, v, indeg)


def _sc_main(src2, code2, dst2, nt_a, nt_b, ct_a, ct_b):
  @functools.partial(
      pl.kernel,
      out_type=jax.ShapeDtypeStruct((NC, NPAD, 192), jnp.float32),
      mesh=_MESH,
      scratch_types=[
          pltpu.VMEM_SHARED((NPAD, 192), jnp.float32),
          pltpu.VMEM((RPT, CW), jnp.int32),
          pltpu.VMEM((RPT, CW), jnp.int32),
          pltpu.VMEM((RPT, CW), jnp.int32),
          pltpu.VMEM((CW, 128), jnp.float32),
          pltpu.VMEM((CW, 128), jnp.float32),
          pltpu.VMEM((CW, 192), jnp.float32),
          pltpu.VMEM((CW, 192), jnp.float32),
          pltpu.VMEM((CW, 192), jnp.float32),
          pltpu.VMEM((CW, 192), jnp.float32),
          pltpu.SemaphoreType.DMA,
          pltpu.SemaphoreType.DMA,
      ],
  )
  def main_kernel(src_hbm, code_hbm, dst_hbm, nta_hbm, ntb_hbm, cta_hbm,
                  ctb_hbm, out_hbm, acc, sidx, cidx, didx, nbufa, cbufa,
                  nbufb, cbufb, obuf, zbuf, gsem1, gsem2):
    c = lax.axis_index("c")
    s = lax.axis_index("s")
    w = c * NS + s

    def zfill(i, _):
      for f in range(192 // 16):
        zbuf[i, pl.ds(f * 16, 16)] = jnp.zeros((16,), jnp.float32)
      return 0
    lax.fori_loop(0, CW, zfill, 0)

    def zacc(t, _):
      pltpu.sync_copy(zbuf, acc.at[pl.ds(s * RN + t * CW, CW)])
      return 0
    lax.fori_loop(0, RN // CW, zacc, 0)

    pltpu.sync_copy(zbuf, obuf)

    base = w * RPT
    pltpu.sync_copy(src_hbm.at[pl.ds(base, RPT)], sidx)
    pltpu.sync_copy(code_hbm.at[pl.ds(base, RPT)], cidx)
    pltpu.sync_copy(dst_hbm.at[pl.ds(base, RPT)], didx)
    plsc.subcore_barrier()

    @pl.when(c == 0)
    def _():
      def chunk(j, _):
        cp1 = pltpu.async_copy(nta_hbm.at[sidx.at[j]], nbufa, gsem1)
        cp2 = pltpu.async_copy(cta_hbm.at[cidx.at[j]], cbufa, gsem2)
        cp1.wait()
        cp2.wait()

        def edge(e, _):
          for f in range(8):
            d = pl.ds(f * 16, 16)
            obuf[e, d] = nbufa[e, d] * cbufa[e, d]
          return 0
        lax.fori_loop(0, CW, edge, 0)
        pltpu.sync_copy(obuf, acc.at[didx.at[j]], add=True)
        return 0
      lax.fori_loop(0, RPT, chunk, 0)

    @pl.when(c == 1)
    def _():
      def chunk(j, _):
        cp1 = pltpu.async_copy(ntb_hbm.at[sidx.at[j]], nbufb, gsem1)
        cp2 = pltpu.async_copy(ctb_hbm.at[cidx.at[j]], cbufb, gsem2)
        cp1.wait()
        cp2.wait()

        def edge(e, _):
          for f in range(12):
            d = pl.ds(f * 16, 16)
            obuf[e, d] = nbufb[e, d] + cbufb[e, d]
          return 0
        lax.fori_loop(0, CW, edge, 0)
        pltpu.sync_copy(obuf, acc.at[didx.at[j]], add=True)
        return 0
      lax.fori_loop(0, RPT, chunk, 0)

    plsc.subcore_barrier()
    pltpu.sync_copy(acc.at[pl.ds(s * RN, RN)],
                    out_hbm.at[c, pl.ds(s * RN, RN)])

  return main_kernel(src2, code2, dst2, nt_a, nt_b, ct_a, ct_b)


def kernel(feat, edge_index, edge_weight, w1, w2, v, bond_tables):
  n = feat.shape[0]
  e = edge_index.shape[1]

  src = edge_index[0]
  dst = edge_index[1]
  pad_idx = jnp.full((EPAD - e,), NPAD - 1, jnp.int32)
  src2 = jnp.concatenate([src, pad_idx]).reshape(ER, CW)
  dst2 = jnp.concatenate([dst, pad_idx]).reshape(ER, CW)
  ewT = jnp.concatenate(
      [edge_weight, jnp.zeros((EPAD - e, 4), jnp.int32)]).T

  featp = jnp.pad(feat, ((0, NPAD - n), (0, 0)))

  degs = _sc_degrees(src2, dst2)
  outdeg = degs[0].reshape(NPAD, 1)
  indeg = degs[1].reshape(NPAD, 1)

  a1, logp, negppk = _tc_node(featp, w1, w2[:128], w2[128:129], outdeg)
  ew, lew, negewpk = _tc_combo(bond_tables)
  codes = _tc_codes(ewT).reshape(ER, CW)

  nt_b = jnp.concatenate([logp, negppk], axis=1)
  ct_b = jnp.concatenate([lew, negewpk], axis=1)

  h = _sc_main(src2, codes, dst2, a1, nt_b, ew, ct_b)
  hs = h[0][:, :128]
  ml = h[1][:, :128]
  pk = h[1][:, 128:]

  rst = _tc_post(hs, ml, pk, v, indeg)
  return rst[:n]


# speedup vs baseline: 7.1859x; 7.1859x over previous
import functools

import jax
import jax.numpy as jnp
from jax import lax
from jax.experimental import pallas as pl
from jax.experimental.pallas import tpu as pltpu
from jax.experimental.pallas import tpu_sc as plsc

NPAD = 10240
EPAD = 163840
CW = 64
ER = EPAD // CW
NC, NS = 2, 16
RPC = ER // NS
STG = 40
RN = NPAD // NS
RADIX = 16384.0
LCLAMP = -30.0

_MESH = plsc.VectorSubcoreMesh(
    core_axis_name="c", subcore_axis_name="s", num_cores=NC, num_subcores=NS)


def _sc_degrees(src2, dst2):
  @functools.partial(
      pl.kernel,
      out_type=jax.ShapeDtypeStruct((NC, NPAD, 128), jnp.float32),
      mesh=_MESH,
      scratch_types=[
          pltpu.VMEM_SHARED((NPAD, 128), jnp.float32),
          pltpu.VMEM((RPC, CW), jnp.int32),
          pltpu.VMEM((CW, 128), jnp.float32),
          pltpu.VMEM((CW, 128), jnp.float32),
      ],
  )
  def deg_kernel(src_hbm, dst_hbm, out_hbm, acc, idx, ones, zbuf):
    c = lax.axis_index("c")
    s = lax.axis_index("s")

    def fill(i, _):
      for f in range(8):
        d = pl.ds(f * 16, 16)
        ones[i, d] = jnp.full((16,), 1.0, jnp.float32)
        zbuf[i, d] = jnp.zeros((16,), jnp.float32)
      return 0
    lax.fori_loop(0, CW, fill, 0)

    def zacc(t, _):
      pltpu.sync_copy(zbuf, acc.at[pl.ds(s * RN + t * CW, CW)])
      return 0
    lax.fori_loop(0, RN // CW, zacc, 0)
    plsc.subcore_barrier()

    def scan(idx_hbm):
      pltpu.sync_copy(idx_hbm.at[pl.ds(s * RPC, RPC)], idx)

      def row(j, _):
        pltpu.sync_copy(ones, acc.at[idx.at[j]], add=True)
        return 0
      lax.fori_loop(0, RPC, row, 0)

    @pl.when(c == 0)
    def _():
      scan(src_hbm)

    @pl.when(c == 1)
    def _():
      scan(dst_hbm)

    plsc.subcore_barrier()
    pltpu.sync_copy(acc.at[pl.ds(s * RN, RN)],
                    out_hbm.at[c, pl.ds(s * RN, RN)])

  return deg_kernel(src2, dst2)


def _tc_node(featp, w1, w2a, b2, outdeg):
  blk = 512
  grid = NPAD // blk

  def body(x_ref, w1_ref, w2a_ref, b2_ref, od_ref, a1_ref, l_ref):
    x = x_ref[...]
    s = lax.rsqrt(jnp.clip(od_ref[...][:, 0:1], 1.0, None))
    a1 = jnp.dot(x, w1_ref[...], preferred_element_type=jnp.float32) * s
    z = jnp.dot(x, w2a_ref[...], preferred_element_type=jnp.float32) * s
    p = jnp.tanh(z + b2_ref[...])
    a1_ref[...] = a1
    negp = jnp.where(p < 0, 1.0, 0.0).astype(jnp.float32)
    l_ref[...] = jnp.maximum(jnp.log(jnp.abs(p)), LCLAMP) + RADIX * negp

  return pl.pallas_call(
      body,
      grid=(grid,),
      in_specs=[
          pl.BlockSpec((blk, 128), lambda i: (i, 0)),
          pl.BlockSpec((128, 128), lambda i: (0, 0)),
          pl.BlockSpec((128, 128), lambda i: (0, 0)),
          pl.BlockSpec((1, 128), lambda i: (0, 0)),
          pl.BlockSpec((blk, 128), lambda i: (i, 0)),
      ],
      out_specs=[
          pl.BlockSpec((blk, 128), lambda i: (i, 0)),
          pl.BlockSpec((blk, 128), lambda i: (i, 0)),
      ],
      out_shape=[
          jax.ShapeDtypeStruct((NPAD, 128), jnp.float32),
          jax.ShapeDtypeStruct((NPAD, 128), jnp.float32),
      ],
  )(featp, w1, w2a, b2, outdeg)


def _tc_combo(bond_tables):
  def body(bt_ref, ew_ref, l_ref):
    bt = bt_ref[...]
    t01 = (bt[0][:, None, :] + bt[1][None, :, :]).reshape(64, 128)
    t012 = (t01[:, None, :] + bt[2][None, :, :]).reshape(512, 128)
    ew = (t012[:, None, :] + bt[3][None, :, :]).reshape(4096, 128)
    ew_ref[...] = ew
    neg = jnp.where(ew < 0, 1.0, 0.0).astype(jnp.float32)
    l_ref[...] = jnp.maximum(jnp.log(jnp.abs(ew)), LCLAMP) + RADIX * neg

  return pl.pallas_call(
      body,
      out_shape=[
          jax.ShapeDtypeStruct((4096, 128), jnp.float32),
          jax.ShapeDtypeStruct((4096, 128), jnp.float32),
      ],
  )(bond_tables)


def _tc_codes(ewT):
  def body(e_ref, c_ref):
    ev = e_ref[...]
    c_ref[...] = ev[0:1] * 512 + ev[1:2] * 64 + ev[2:3] * 8 + ev[3:4]

  return pl.pallas_call(
      body,
      out_shape=jax.ShapeDtypeStruct((1, EPAD), jnp.int32),
  )(ewT)


def _tc_post(hs, sv, v, indeg):
  blk = 512
  grid = NPAD // blk

  def body(hs_ref, s_ref, v_ref, id_ref, out_ref):
    sval = s_ref[...]
    cnt = jnp.floor(sval * (1.0 / RADIX) + 0.5)
    lg = sval - RADIX * cnt
    par = cnt - 2.0 * jnp.floor(cnt * 0.5)
    hp = (1.0 - 2.0 * par) * jnp.exp(lg)
    r = hs_ref[...] + jnp.dot(hp, v_ref[...], preferred_element_type=jnp.float32)
    nd = lax.rsqrt(jnp.clip(id_ref[...][:, 0:1], 1.0, None))
    out_ref[...] = r * nd

  return pl.pallas_call(
      body,
      grid=(grid,),
      in_specs=[
          pl.BlockSpec((blk, 128), lambda i: (i, 0)),
          pl.BlockSpec((blk, 128), lambda i: (i, 0)),
          pl.BlockSpec((128, 128), lambda i: (0, 0)),
          pl.BlockSpec((blk, 128), lambda i: (i, 0)),
      ],
      out_specs=pl.BlockSpec((blk, 128), lambda i: (i, 0)),
      out_shape=jax.ShapeDtypeStruct((NPAD, 128), jnp.float32),
  )(hs, sv, v, indeg)


def _sc_main(src2, code2, dst2, nt_a, nt_b, ct_a, ct_b):
  @functools.partial(
      pl.kernel,
      out_type=jax.ShapeDtypeStruct((NC, NPAD, 128), jnp.float32),
      mesh=_MESH,
      scratch_types=[
          pltpu.VMEM_SHARED((NPAD, 128), jnp.float32),
          pltpu.VMEM((STG, CW), jnp.int32),
          pltpu.VMEM((STG, CW), jnp.int32),
          pltpu.VMEM((STG, CW), jnp.int32),
          pltpu.VMEM((CW, 128), jnp.float32),
          pltpu.VMEM((CW, 128), jnp.float32),
          pltpu.VMEM((CW, 128), jnp.float32),
          pltpu.SemaphoreType.DMA,
          pltpu.SemaphoreType.DMA,
      ],
  )
  def main_kernel(src_hbm, code_hbm, dst_hbm, nta_hbm, ntb_hbm, cta_hbm,
                  ctb_hbm, out_hbm, acc, sidx, cidx, didx, nbuf, cbuf,
                  obuf, gsem1, gsem2):
    c = lax.axis_index("c")
    s = lax.axis_index("s")

    def zfill(i, _):
      for f in range(8):
        obuf[i, pl.ds(f * 16, 16)] = jnp.zeros((16,), jnp.float32)
      return 0
    lax.fori_loop(0, CW, zfill, 0)

    def zacc(t, _):
      pltpu.sync_copy(obuf, acc.at[pl.ds(s * RN + t * CW, CW)])
      return 0
    lax.fori_loop(0, RN // CW, zacc, 0)
    plsc.subcore_barrier()

    def run(node_tbl, combo_tbl, is_mul):
      def stage(t, _):
        base = s * RPC + t * STG
        pltpu.sync_copy(src_hbm.at[pl.ds(base, STG)], sidx)
        pltpu.sync_copy(code_hbm.at[pl.ds(base, STG)], cidx)
        pltpu.sync_copy(dst_hbm.at[pl.ds(base, STG)], didx)

        def chunk(j, _):
          cp1 = pltpu.async_copy(node_tbl.at[sidx.at[j]], nbuf, gsem1)
          cp2 = pltpu.async_copy(combo_tbl.at[cidx.at[j]], cbuf, gsem2)
          cp1.wait()
          cp2.wait()

          def edge(e, _):
            for f in range(8):
              d = pl.ds(f * 16, 16)
              if is_mul:
                obuf[e, d] = nbuf[e, d] * cbuf[e, d]
              else:
                obuf[e, d] = nbuf[e, d] + cbuf[e, d]
            return 0
          lax.fori_loop(0, CW, edge, 0)
          pltpu.sync_copy(obuf, acc.at[didx.at[j]], add=True)
          return 0
        lax.fori_loop(0, STG, chunk, 0)
        return 0
      lax.fori_loop(0, RPC // STG, stage, 0)

    @pl.when(c == 0)
    def _():
      run(nta_hbm, cta_hbm, True)

    @pl.when(c == 1)
    def _():
      run(ntb_hbm, ctb_hbm, False)

    plsc.subcore_barrier()
    pltpu.sync_copy(acc.at[pl.ds(s * RN, RN)],
                    out_hbm.at[c, pl.ds(s * RN, RN)])

  return main_kernel(src2, code2, dst2, nt_a, nt_b, ct_a, ct_b)


def kernel(feat, edge_index, edge_weight, w1, w2, v, bond_tables):
  n = feat.shape[0]
  e = edge_index.shape[1]

  src = edge_index[0]
  dst = edge_index[1]
  pad_idx = jnp.full((EPAD - e,), NPAD - 1, jnp.int32)
  src2 = jnp.concatenate([src, pad_idx]).reshape(ER, CW)
  dst2 = jnp.concatenate([dst, pad_idx]).reshape(ER, CW)
  ewT = jnp.concatenate(
      [edge_weight, jnp.zeros((EPAD - e, 4), jnp.int32)]).T

  featp = jnp.pad(feat, ((0, NPAD - n), (0, 0)))

  degs = _sc_degrees(src2, dst2)

  a1, nt_b = _tc_node(featp, w1, w2[:128], w2[128:129], degs[0])
  ew, ct_b = _tc_combo(bond_tables)
  codes = _tc_codes(ewT).reshape(ER, CW)

  h = _sc_main(src2, codes, dst2, a1, nt_b, ew, ct_b)

  rst = _tc_post(h[0], h[1], v, degs[1])
  return rst[:n]


# speedup vs baseline: 9.2038x; 1.2808x over previous
import functools

import jax
import jax.numpy as jnp
from jax import lax
from jax.experimental import pallas as pl
from jax.experimental.pallas import tpu as pltpu
from jax.experimental.pallas import tpu_sc as plsc

NPAD = 10240
EPAD = 163840
CW = 64
ER = EPAD // CW
NC, NS = 2, 16
RPC = ER // NS
STG = 40
RN = NPAD // NS
RADIX = 16384.0
LCLAMP = -30.0

_MESH = plsc.VectorSubcoreMesh(
    core_axis_name="c", subcore_axis_name="s", num_cores=NC, num_subcores=NS)


def _sc_degrees(src2, dst2):
  @functools.partial(
      pl.kernel,
      out_type=jax.ShapeDtypeStruct((NC, NPAD, 128), jnp.float32),
      mesh=_MESH,
      scratch_types=[
          pltpu.VMEM_SHARED((NPAD, 128), jnp.float32),
          pltpu.VMEM((RPC, CW), jnp.int32),
          pltpu.VMEM((CW, 128), jnp.float32),
          pltpu.VMEM((CW, 128), jnp.float32),
      ],
  )
  def deg_kernel(src_hbm, dst_hbm, out_hbm, acc, idx, ones, zbuf):
    c = lax.axis_index("c")
    s = lax.axis_index("s")

    def fill(i, _):
      for f in range(8):
        d = pl.ds(f * 16, 16)
        ones[i, d] = jnp.full((16,), 1.0, jnp.float32)
        zbuf[i, d] = jnp.zeros((16,), jnp.float32)
      return 0
    lax.fori_loop(0, CW, fill, 0)

    def zacc(t, _):
      pltpu.sync_copy(zbuf, acc.at[pl.ds(s * RN + t * CW, CW)])
      return 0
    lax.fori_loop(0, RN // CW, zacc, 0)
    plsc.subcore_barrier()

    def scan(idx_hbm):
      pltpu.sync_copy(idx_hbm.at[pl.ds(s * RPC, RPC)], idx)

      def row(j, _):
        pltpu.sync_copy(ones, acc.at[idx.at[j]], add=True)
        return 0
      lax.fori_loop(0, RPC, row, 0)

    @pl.when(c == 0)
    def _():
      scan(src_hbm)

    @pl.when(c == 1)
    def _():
      scan(dst_hbm)

    plsc.subcore_barrier()
    pltpu.sync_copy(acc.at[pl.ds(s * RN, RN)],
                    out_hbm.at[c, pl.ds(s * RN, RN)])

  return deg_kernel(src2, dst2)


def _tc_node(featp, w1, w2a, b2, outdeg):
  blk = 512
  grid = NPAD // blk

  def body(x_ref, w1_ref, w2a_ref, b2_ref, od_ref, a1_ref, l_ref):
    x = x_ref[...]
    s = lax.rsqrt(jnp.clip(od_ref[...][:, 0:1], 1.0, None))
    a1 = jnp.dot(x, w1_ref[...], preferred_element_type=jnp.float32) * s
    z = jnp.dot(x, w2a_ref[...], preferred_element_type=jnp.float32) * s
    p = jnp.tanh(z + b2_ref[...])
    a1_ref[...] = a1
    negp = jnp.where(p < 0, 1.0, 0.0).astype(jnp.float32)
    l_ref[...] = jnp.maximum(jnp.log(jnp.abs(p)), LCLAMP) + RADIX * negp

  return pl.pallas_call(
      body,
      grid=(grid,),
      in_specs=[
          pl.BlockSpec((blk, 128), lambda i: (i, 0)),
          pl.BlockSpec((128, 128), lambda i: (0, 0)),
          pl.BlockSpec((128, 128), lambda i: (0, 0)),
          pl.BlockSpec((1, 128), lambda i: (0, 0)),
          pl.BlockSpec((blk, 128), lambda i: (i, 0)),
      ],
      out_specs=[
          pl.BlockSpec((blk, 128), lambda i: (i, 0)),
          pl.BlockSpec((blk, 128), lambda i: (i, 0)),
      ],
      out_shape=[
          jax.ShapeDtypeStruct((NPAD, 128), jnp.float32),
          jax.ShapeDtypeStruct((NPAD, 128), jnp.float32),
      ],
  )(featp, w1, w2a, b2, outdeg)


def _tc_combo(bond_tables):
  def body(bt_ref, ew_ref, l_ref):
    bt = bt_ref[...]
    t01 = (bt[0][:, None, :] + bt[1][None, :, :]).reshape(64, 128)
    t012 = (t01[:, None, :] + bt[2][None, :, :]).reshape(512, 128)
    ew = (t012[:, None, :] + bt[3][None, :, :]).reshape(4096, 128)
    ew_ref[...] = ew
    neg = jnp.where(ew < 0, 1.0, 0.0).astype(jnp.float32)
    l_ref[...] = jnp.maximum(jnp.log(jnp.abs(ew)), LCLAMP) + RADIX * neg

  return pl.pallas_call(
      body,
      out_shape=[
          jax.ShapeDtypeStruct((4096, 128), jnp.float32),
          jax.ShapeDtypeStruct((4096, 128), jnp.float32),
      ],
  )(bond_tables)


def _tc_codes(ewT):
  def body(e_ref, c_ref):
    ev = e_ref[...]
    c_ref[...] = ev[0:1] * 512 + ev[1:2] * 64 + ev[2:3] * 8 + ev[3:4]

  return pl.pallas_call(
      body,
      out_shape=jax.ShapeDtypeStruct((1, EPAD), jnp.int32),
  )(ewT)


def _tc_post(hs, sv, v, indeg):
  blk = 512
  grid = NPAD // blk

  def body(hs_ref, s_ref, v_ref, id_ref, out_ref):
    sval = s_ref[...]
    cnt = jnp.floor(sval * (1.0 / RADIX) + 0.5)
    lg = sval - RADIX * cnt
    par = cnt - 2.0 * jnp.floor(cnt * 0.5)
    hp = (1.0 - 2.0 * par) * jnp.exp(lg)
    r = hs_ref[...] + jnp.dot(hp, v_ref[...], preferred_element_type=jnp.float32)
    nd = lax.rsqrt(jnp.clip(id_ref[...][:, 0:1], 1.0, None))
    out_ref[...] = r * nd

  return pl.pallas_call(
      body,
      grid=(grid,),
      in_specs=[
          pl.BlockSpec((blk, 128), lambda i: (i, 0)),
          pl.BlockSpec((blk, 128), lambda i: (i, 0)),
          pl.BlockSpec((128, 128), lambda i: (0, 0)),
          pl.BlockSpec((blk, 128), lambda i: (i, 0)),
      ],
      out_specs=pl.BlockSpec((blk, 128), lambda i: (i, 0)),
      out_shape=jax.ShapeDtypeStruct((NPAD, 128), jnp.float32),
  )(hs, sv, v, indeg)


def _sc_main(src2, code2, dst2, nt_a, nt_b, ct_a, ct_b):
  @functools.partial(
      pl.kernel,
      out_type=jax.ShapeDtypeStruct((NC, NPAD, 128), jnp.float32),
      mesh=_MESH,
      scratch_types=[
          pltpu.VMEM_SHARED((NPAD, 128), jnp.float32),
          pltpu.VMEM((STG, CW), jnp.int32),
          pltpu.VMEM((STG, CW), jnp.int32),
          pltpu.VMEM((STG, CW), jnp.int32),
          pltpu.VMEM((CW, 128), jnp.float32),
          pltpu.VMEM((CW, 128), jnp.float32),
          pltpu.VMEM((CW, 128), jnp.float32),
          pltpu.VMEM((CW, 128), jnp.float32),
          pltpu.SemaphoreType.DMA,
          pltpu.SemaphoreType.DMA,
          pltpu.SemaphoreType.DMA,
          pltpu.SemaphoreType.DMA,
          pltpu.SemaphoreType.DMA,
          pltpu.SemaphoreType.DMA,
      ],
  )
  def main_kernel(src_hbm, code_hbm, dst_hbm, nta_hbm, ntb_hbm, cta_hbm,
                  ctb_hbm, out_hbm, acc, sidx, cidx, didx, nbuf0, nbuf1,
                  cbuf0, cbuf1, gn0, gn1, gc0, gc1, ss0, ss1):
    c = lax.axis_index("c")
    s = lax.axis_index("s")
    nb = [nbuf0, nbuf1]
    cb = [cbuf0, cbuf1]
    gn = [gn0, gn1]
    gc = [gc0, gc1]
    ssem = [ss0, ss1]

    def zfill(i, _):
      for f in range(8):
        nbuf0[i, pl.ds(f * 16, 16)] = jnp.zeros((16,), jnp.float32)
      return 0
    lax.fori_loop(0, CW, zfill, 0)

    def zacc(t, _):
      pltpu.sync_copy(nbuf0, acc.at[pl.ds(s * RN + t * CW, CW)])
      return 0
    lax.fori_loop(0, RN // CW, zacc, 0)
    plsc.subcore_barrier()

    def run(node_tbl, combo_tbl, is_mul):
      def stage(t, _):
        base = s * RPC + t * STG
        pltpu.sync_copy(src_hbm.at[pl.ds(base, STG)], sidx)
        pltpu.sync_copy(code_hbm.at[pl.ds(base, STG)], cidx)
        pltpu.sync_copy(dst_hbm.at[pl.ds(base, STG)], didx)

        gath = {}
        scat = [None, None]
        gath[0] = (pltpu.async_copy(node_tbl.at[sidx.at[0]], nb[0], gn[0]),
                   pltpu.async_copy(combo_tbl.at[cidx.at[0]], cb[0], gc[0]))
        for j in range(STG):
          pj = j % 2
          if j + 1 < STG:
            nx = (j + 1) % 2
            if scat[nx] is not None:
              scat[nx].wait()
              scat[nx] = None
            gath[j + 1] = (
                pltpu.async_copy(node_tbl.at[sidx.at[j + 1]], nb[nx], gn[nx]),
                pltpu.async_copy(combo_tbl.at[cidx.at[j + 1]], cb[nx], gc[nx]))
          gath[j][0].wait()
          gath[j][1].wait()
          nbj = nb[pj]
          cbj = cb[pj]

          def edge(e, _, nbj=nbj, cbj=cbj):
            for f in range(8):
              d = pl.ds(f * 16, 16)
              if is_mul:
                nbj[e, d] = nbj[e, d] * cbj[e, d]
              else:
                nbj[e, d] = nbj[e, d] + cbj[e, d]
            return 0
          lax.fori_loop(0, CW, edge, 0)
          scat[pj] = pltpu.async_copy(nbj, acc.at[didx.at[j]], ssem[pj],
                                      add=True)
        for d_ in scat:
          if d_ is not None:
            d_.wait()
        return 0
      lax.fori_loop(0, RPC // STG, stage, 0)

    @pl.when(c == 0)
    def _():
      run(nta_hbm, cta_hbm, True)

    @pl.when(c == 1)
    def _():
      run(ntb_hbm, ctb_hbm, False)

    plsc.subcore_barrier()
    pltpu.sync_copy(acc.at[pl.ds(s * RN, RN)],
                    out_hbm.at[c, pl.ds(s * RN, RN)])

  return main_kernel(src2, code2, dst2, nt_a, nt_b, ct_a, ct_b)


def kernel(feat, edge_index, edge_weight, w1, w2, v, bond_tables):
  n = feat.shape[0]
  e = edge_index.shape[1]

  src = edge_index[0]
  dst = edge_index[1]
  pad_idx = jnp.full((EPAD - e,), NPAD - 1, jnp.int32)
  src2 = jnp.concatenate([src, pad_idx]).reshape(ER, CW)
  dst2 = jnp.concatenate([dst, pad_idx]).reshape(ER, CW)
  ewT = jnp.concatenate(
      [edge_weight, jnp.zeros((EPAD - e, 4), jnp.int32)]).T

  featp = jnp.pad(feat, ((0, NPAD - n), (0, 0)))

  degs = _sc_degrees(src2, dst2)

  a1, nt_b = _tc_node(featp, w1, w2[:128], w2[128:129], degs[0])
  ew, ct_b = _tc_combo(bond_tables)
  codes = _tc_codes(ewT).reshape(ER, CW)

  h = _sc_main(src2, codes, dst2, a1, nt_b, ew, ct_b)

  rst = _tc_post(h[0], h[1], v, degs[1])
  return rst[:n]
